# baseline (speedup 1.0000x reference)
import functools

import jax
import jax.numpy as jnp
from jax import lax
from jax.experimental import pallas as pl
from jax.experimental.pallas import tpu as pltpu
from jax.experimental.pallas import tpu_sc as plsc

N_NODES = 10000
N_EDGES = 320000
D_FEAT = 128
D_EDGE = 16

NC = 2
NS = 16
NW = NC * NS
CW = 128
ROWS = N_EDGES // CW
B0 = 72
B1 = 8
B2 = 4
N_PAD = 10240
ZSLICE = N_PAD // NS

_GRID = 20
_EB = 16384
ROWS_PAD = _GRID * _EB // CW


def _msg_kernel(attr_ref, wm_ref, s_ref, msg_ref):
    m = jnp.sum(attr_ref[...] * wm_ref[...], axis=1) + s_ref[2]
    msg_ref[...] = m.reshape(_EB // CW, CW)


_msg_call = pl.pallas_call(
    _msg_kernel,
    grid=(_GRID,),
    in_specs=[
        pl.BlockSpec((_EB, D_EDGE), lambda i: (i, 0)),
        pl.BlockSpec((1, D_EDGE), lambda i: (0, 0)),
        pl.BlockSpec(memory_space=pltpu.SMEM),
    ],
    out_specs=pl.BlockSpec((_EB // CW, CW), lambda i: (i, 0)),
    out_shape=jax.ShapeDtypeStruct((ROWS_PAD, CW), jnp.float32),
)


_mesh = plsc.VectorSubcoreMesh(core_axis_name="c", subcore_axis_name="s")


@functools.partial(
    pl.kernel,
    mesh=_mesh,
    out_type=jax.ShapeDtypeStruct((NC, N_PAD), jnp.float32),
    scratch_types=[
        pltpu.VMEM((B0, CW), jnp.int32),
        pltpu.VMEM((B0, CW), jnp.float32),
        pltpu.VMEM((B1, CW), jnp.int32),
        pltpu.VMEM((B1, CW), jnp.float32),
        pltpu.VMEM((B2, CW), jnp.int32),
        pltpu.VMEM((B2, CW), jnp.float32),
        pltpu.VMEM((ZSLICE,), jnp.float32),
        pltpu.VMEM_SHARED((N_PAD,), jnp.float32),
    ],
)
def _scatter_call(dst_hbm, msg_hbm, out_hbm,
                  idx_v, msg_v, idx_x, msg_x, idx_t, msg_t, zbuf, aggr_sh):
    c = lax.axis_index("c")
    s = lax.axis_index("s")
    wid = c * NS + s
    start = pl.multiple_of(wid * B0 + B1 * jnp.minimum(wid, 24), 8)
    start2 = pl.multiple_of(start + B0, 8)
    has_extra = wid < 24
    is_tail = wid == NW - 1
    for j in range(ZSLICE // 16):
        zbuf[pl.ds(j * 16, 16)] = jnp.zeros((16,), jnp.float32)
    pltpu.sync_copy(zbuf, aggr_sh.at[pl.ds(s * ZSLICE, ZSLICE)])
    pltpu.sync_copy(dst_hbm.at[pl.ds(start, B0)], idx_v)
    pltpu.sync_copy(msg_hbm.at[pl.ds(start, B0)], msg_v)

    @pl.when(has_extra)
    def _():
        pltpu.sync_copy(dst_hbm.at[pl.ds(start2, B1)], idx_x)
        pltpu.sync_copy(msg_hbm.at[pl.ds(start2, B1)], msg_x)

    @pl.when(is_tail)
    def _():
        pltpu.sync_copy(dst_hbm.at[pl.ds(start2, B2)], idx_t)
        pltpu.sync_copy(msg_hbm.at[pl.ds(start2, B2)], msg_t)

    plsc.subcore_barrier()

    def body(j, carry):
        pltpu.sync_copy(msg_v.at[j], aggr_sh.at[idx_v.at[j]], add=True)
        return carry

    lax.fori_loop(0, B0, body, 0)

    @pl.when(has_extra)
    def _():
        def bodyx(j, carry):
            pltpu.sync_copy(msg_x.at[j], aggr_sh.at[idx_x.at[j]], add=True)
            return carry
        lax.fori_loop(0, B1, bodyx, 0)

    @pl.when(is_tail)
    def _():
        def bodyt(j, carry):
            pltpu.sync_copy(msg_t.at[j], aggr_sh.at[idx_t.at[j]], add=True)
            return carry
        lax.fori_loop(0, B2, bodyt, 0)

    plsc.subcore_barrier()

    @pl.when(s == 0)
    def _():
        pltpu.sync_copy(aggr_sh, out_hbm.at[c])


def _upd_kernel(x_ref, a0_ref, a1_ref, wu_ref, s_ref, out_ref):
    xb = x_ref[...]
    r = jnp.sum(xb * wu_ref[...], axis=1, keepdims=True)
    aggr = a0_ref[...] + a1_ref[...]
    out_ref[...] = xb[:, 0:1] + r + aggr * xb[:, 3:4] * s_ref[0] + s_ref[1]


_upd_call = pl.pallas_call(
    _upd_kernel,
    grid=(10,),
    in_specs=[
        pl.BlockSpec((1000, 128), lambda i: (i, 0)),
        pl.BlockSpec((1000, 1), lambda i: (i, 0)),
        pl.BlockSpec((1000, 1), lambda i: (i, 0)),
        pl.BlockSpec((1, 128), lambda i: (0, 0)),
        pl.BlockSpec(memory_space=pltpu.SMEM),
    ],
    out_specs=pl.BlockSpec((1000, 1), lambda i: (i, 0)),
    out_shape=jax.ShapeDtypeStruct((N_NODES, 1), jnp.float32),
)


def kernel(x, edge_index, edge_attr, W_msg, b_msg, W_upd, b_upd):
    wm = W_msg.reshape(1, D_EDGE)
    scal = jnp.stack([W_upd[D_FEAT, 0], b_upd[0], b_msg[0]])

    msg = _msg_call(edge_attr, wm, scal)

    dst2d = edge_index[1].astype(jnp.int32).reshape(ROWS, CW)
    aggr2 = _scatter_call(dst2d, msg)

    a0 = aggr2[0, :N_NODES].reshape(N_NODES, 1)
    a1 = aggr2[1, :N_NODES].reshape(N_NODES, 1)
    w_vec = W_upd[:D_FEAT].reshape(1, D_FEAT)
    return _upd_call(x, a0, a1, w_vec, scal)
